# Initial kernel scaffold; baseline (speedup 1.0000x reference)
import jax
import jax.numpy as jnp
from jax.experimental import pallas as pl


def kernel(nodes, pca_matrix, W, b):
    raise NotImplementedError("write your pallas kernel here")


# speedup vs baseline: 3.1321x; 3.1321x over previous
import functools

import jax
import jax.numpy as jnp
from jax import lax
from jax.experimental import pallas as pl
from jax.experimental.pallas import tpu as pltpu
from jax.experimental.pallas import tpu_sc as plsc

NC = 2
NS = 16
NW = NC * NS


_ROW_BLK = 2048


def _transform_body(pca_ref, wt_ref, b_ref, out_ref):
    out_ref[...] = (
        jnp.dot(pca_ref[...], wt_ref[...],
                preferred_element_type=jnp.float32,
                precision=lax.Precision.HIGHEST)
        + b_ref[...]
    )


def _transform_table(pca_matrix, wt, b2d):
    v, d = pca_matrix.shape
    grid = (v + _ROW_BLK - 1) // _ROW_BLK
    return pl.pallas_call(
        _transform_body,
        grid=(grid,),
        in_specs=[
            pl.BlockSpec((_ROW_BLK, d), lambda i: (i, 0)),
            pl.BlockSpec((d, d), lambda i: (0, 0)),
            pl.BlockSpec((1, d), lambda i: (0, 0)),
        ],
        out_specs=pl.BlockSpec((_ROW_BLK, d), lambda i: (i, 0)),
        out_shape=jax.ShapeDtypeStruct((v, d), jnp.float32),
    )(pca_matrix, wt, b2d)


_CHUNK = 128


def _make_gather(n_flat, d):
    per_w = n_flat // NW
    n_chunks = per_w // _CHUNK
    mesh = plsc.VectorSubcoreMesh(
        core_axis_name="c", subcore_axis_name="s",
        num_cores=NC, num_subcores=NS)

    @functools.partial(
        pl.kernel,
        out_type=jax.ShapeDtypeStruct((n_flat, d), jnp.float32),
        mesh=mesh,
        scratch_types=[
            pltpu.VMEM((n_chunks, _CHUNK), jnp.int32),
            pltpu.VMEM((_CHUNK, d), jnp.float32),
            pltpu.SemaphoreType.DMA,
        ],
        compiler_params=pltpu.CompilerParams(use_tc_tiling_on_sc=False),
    )
    def gather(table_hbm, idx_hbm, out_hbm, idx_v, rows_v, sem):
        wid = lax.axis_index("s") * NC + lax.axis_index("c")
        pltpu.sync_copy(idx_hbm.at[wid], idx_v)
        base = wid * per_w

        def body(j, carry):
            pltpu.async_copy(table_hbm.at[idx_v.at[j]], rows_v, sem).wait()
            pltpu.sync_copy(rows_v, out_hbm.at[pl.ds(base + j * _CHUNK, _CHUNK)])
            return carry

        lax.fori_loop(0, n_chunks, body, 0)

    return gather


def kernel(nodes, pca_matrix, W, b):
    bsz, seq = nodes.shape
    d = pca_matrix.shape[1]
    n_flat = bsz * seq

    table = _transform_table(pca_matrix, W.T, b.reshape(1, d))

    per_w = n_flat // NW
    idx = nodes.reshape(-1).astype(jnp.int32).reshape(NW, per_w // _CHUNK, _CHUNK)
    out = _make_gather(n_flat, d)(table, idx)
    return out.reshape(bsz, seq, d)


# speedup vs baseline: 3.6970x; 1.1804x over previous
import functools

import jax
import jax.numpy as jnp
from jax import lax
from jax.experimental import pallas as pl
from jax.experimental.pallas import tpu as pltpu
from jax.experimental.pallas import tpu_sc as plsc

NC = 2
NS = 16
NW = NC * NS


_ROW_BLK = 2048


def _transform_body(pca_ref, wt_ref, b_ref, out_ref):
    out_ref[...] = (
        jnp.dot(pca_ref[...], wt_ref[...],
                preferred_element_type=jnp.float32,
                precision=lax.Precision.HIGHEST)
        + b_ref[...]
    )


def _transform_table(pca_matrix, wt, b2d):
    v, d = pca_matrix.shape
    grid = (v + _ROW_BLK - 1) // _ROW_BLK
    return pl.pallas_call(
        _transform_body,
        grid=(grid,),
        in_specs=[
            pl.BlockSpec((_ROW_BLK, d), lambda i: (i, 0)),
            pl.BlockSpec((d, d), lambda i: (0, 0)),
            pl.BlockSpec((1, d), lambda i: (0, 0)),
        ],
        out_specs=pl.BlockSpec((_ROW_BLK, d), lambda i: (i, 0)),
        out_shape=jax.ShapeDtypeStruct((v, d), jnp.float32),
    )(pca_matrix, wt, b2d)


_CHUNK = 128
_K = 4


def _make_gather(n_flat, d):
    per_w = n_flat // NW
    n_chunks = per_w // _CHUNK
    n_groups = n_chunks // _K
    grp = _K * _CHUNK
    mesh = plsc.VectorSubcoreMesh(
        core_axis_name="c", subcore_axis_name="s",
        num_cores=NC, num_subcores=NS)

    @functools.partial(
        pl.kernel,
        out_type=jax.ShapeDtypeStruct((n_flat, d), jnp.float32),
        mesh=mesh,
        scratch_types=[
            pltpu.VMEM((n_chunks, _CHUNK), jnp.int32),
            pltpu.VMEM((2, grp, d), jnp.float32),
            pltpu.SemaphoreType.DMA,
            pltpu.SemaphoreType.DMA,
        ],
        compiler_params=pltpu.CompilerParams(use_tc_tiling_on_sc=False),
    )
    def gather(table_hbm, idx_hbm, out_hbm, idx_v, rows_v, sem0, sem1):
        wid = lax.axis_index("s") * NC + lax.axis_index("c")
        pltpu.sync_copy(idx_hbm.at[wid], idx_v)
        base = wid * per_w
        sems = (sem0, sem1)

        def fire(g, b):
            for k in range(_K):
                pltpu.async_copy(
                    table_hbm.at[idx_v.at[g * _K + k]],
                    rows_v.at[b, pl.ds(k * _CHUNK, _CHUNK)],
                    sems[b])

        def drain(b):
            for k in range(_K):
                pltpu.make_async_copy(
                    table_hbm.at[idx_v.at[0]],
                    rows_v.at[b, pl.ds(k * _CHUNK, _CHUNK)],
                    sems[b]).wait()

        def write(g, b):
            pltpu.sync_copy(rows_v.at[b],
                            out_hbm.at[pl.ds(base + g * grp, grp)])

        fire(0, 0)

        def body(i2, carry):
            i = i2 * 2
            fire(i + 1, 1)
            drain(0)
            write(i, 0)
            fire(i + 2, 0)
            drain(1)
            write(i + 1, 1)
            return carry

        lax.fori_loop(0, n_groups // 2 - 1, body, 0)
        i = n_groups - 2
        fire(i + 1, 1)
        drain(0)
        write(i, 0)
        drain(1)
        write(i + 1, 1)

    return gather


def kernel(nodes, pca_matrix, W, b):
    bsz, seq = nodes.shape
    d = pca_matrix.shape[1]
    n_flat = bsz * seq

    table = _transform_table(pca_matrix, W.T, b.reshape(1, d))

    per_w = n_flat // NW
    idx = nodes.reshape(-1).astype(jnp.int32).reshape(NW, per_w // _CHUNK, _CHUNK)
    out = _make_gather(n_flat, d)(table, idx)
    return out.reshape(bsz, seq, d)


# speedup vs baseline: 4.8961x; 1.3243x over previous
import functools

import jax
import jax.numpy as jnp
from jax import lax
from jax.experimental import pallas as pl
from jax.experimental.pallas import tpu as pltpu
from jax.experimental.pallas import tpu_sc as plsc

NC = 2
NS = 16
NW = NC * NS

DP = 128


_ROW_BLK = 2048


def _transform_body(pca_ref, wt_ref, b_ref, out_ref):
    out_ref[...] = (
        jnp.dot(pca_ref[...], wt_ref[...],
                preferred_element_type=jnp.float32,
                precision=lax.Precision.HIGHEST)
        + b_ref[...]
    )


def _transform_table(pca_matrix, wt, b2d):
    v, d = pca_matrix.shape
    grid = (v + _ROW_BLK - 1) // _ROW_BLK
    return pl.pallas_call(
        _transform_body,
        grid=(grid,),
        in_specs=[
            pl.BlockSpec((_ROW_BLK, d), lambda i: (i, 0)),
            pl.BlockSpec((d, DP), lambda i: (0, 0)),
            pl.BlockSpec((1, DP), lambda i: (0, 0)),
        ],
        out_specs=pl.BlockSpec((_ROW_BLK, DP), lambda i: (i, 0)),
        out_shape=jax.ShapeDtypeStruct((v, DP), jnp.float32),
    )(pca_matrix, wt, b2d)


_CHUNK = 128
_K = 2


def _make_gather(n_flat):
    per_w = n_flat // NW
    n_chunks = per_w // _CHUNK
    n_groups = n_chunks // _K
    grp = _K * _CHUNK
    mesh = plsc.VectorSubcoreMesh(
        core_axis_name="c", subcore_axis_name="s",
        num_cores=NC, num_subcores=NS)

    @functools.partial(
        pl.kernel,
        out_type=jax.ShapeDtypeStruct((n_flat, DP), jnp.float32),
        mesh=mesh,
        scratch_types=[
            pltpu.VMEM((n_chunks, _CHUNK), jnp.int32),
            pltpu.VMEM((2, grp, DP), jnp.float32),
            pltpu.SemaphoreType.DMA,
            pltpu.SemaphoreType.DMA,
        ],
        compiler_params=pltpu.CompilerParams(use_tc_tiling_on_sc=True),
    )
    def gather(table_hbm, idx_hbm, out_hbm, idx_v, rows_v, sem0, sem1):
        wid = lax.axis_index("s") * NC + lax.axis_index("c")
        pltpu.sync_copy(idx_hbm.at[wid], idx_v)
        base = wid * per_w
        sems = (sem0, sem1)

        def fire(g, b):
            for k in range(_K):
                pltpu.async_copy(
                    table_hbm.at[idx_v.at[g * _K + k]],
                    rows_v.at[b, pl.ds(k * _CHUNK, _CHUNK)],
                    sems[b])

        def drain(b):
            for k in range(_K):
                pltpu.make_async_copy(
                    table_hbm.at[idx_v.at[0]],
                    rows_v.at[b, pl.ds(k * _CHUNK, _CHUNK)],
                    sems[b]).wait()

        def write(g, b):
            pltpu.sync_copy(rows_v.at[b],
                            out_hbm.at[pl.ds(base + g * grp, grp)])

        fire(0, 0)

        def body(i2, carry):
            i = i2 * 2
            fire(i + 1, 1)
            drain(0)
            write(i, 0)
            fire(i + 2, 0)
            drain(1)
            write(i + 1, 1)
            return carry

        lax.fori_loop(0, n_groups // 2 - 1, body, 0)
        i = n_groups - 2
        fire(i + 1, 1)
        drain(0)
        write(i, 0)
        drain(1)
        write(i + 1, 1)

    return gather


def kernel(nodes, pca_matrix, W, b):
    bsz, seq = nodes.shape
    d = pca_matrix.shape[1]
    n_flat = bsz * seq

    wt = jnp.zeros((d, DP), jnp.float32).at[:, :d].set(W.T)
    b2d = jnp.zeros((1, DP), jnp.float32).at[:, :d].set(b)
    table = _transform_table(pca_matrix, wt, b2d)

    per_w = n_flat // NW
    idx = nodes.reshape(-1).astype(jnp.int32).reshape(NW, per_w // _CHUNK, _CHUNK)
    out = _make_gather(n_flat)(table, idx)
    return out[:, :d].reshape(bsz, seq, d)
